# Initial kernel scaffold; baseline (speedup 1.0000x reference)
import jax
import jax.numpy as jnp
from jax.experimental import pallas as pl


def kernel(x, edge_attr, edge_fc_w, edge_fc_b, w1l, w1r, b1, w2l, w2r, b2, proj_w, proj_b, edge_index, batch):
    raise NotImplementedError("write your pallas kernel here")


# speedup vs baseline: 1.8030x; 1.8030x over previous
import functools

import jax
import jax.numpy as jnp
from jax import lax
from jax.experimental import pallas as pl
from jax.experimental.pallas import tpu as pltpu
from jax.experimental.pallas import tpu_sc as plsc

N = 100000
E = 1600000
IN = 18
ED = 22
H = 64
OUT = 128
G = 64

NTILE = 16
NP = 100096
RPT = NP // NTILE
K = 80
NCH = E // K

_mesh = plsc.VectorSubcoreMesh(core_axis_name="c", subcore_axis_name="s")
_sc_params = pltpu.CompilerParams(use_tc_tiling_on_sc=False)


_BE = 8000
_NBE = E // _BE


def _edge_mm_body(attr_ref, w_ref, b_ref, out_ref, eu_ref):
    p = pl.program_id(1)

    @pl.when(p == 0)
    def _():
        eu_ref[...] = (
            jnp.dot(attr_ref[...], w_ref[...],
                    preferred_element_type=jnp.float32)
            + b_ref[...]
        )
        out_ref[...] = eu_ref[:, 0:16]

    @pl.when(p == 1)
    def _():
        out_ref[...] = jnp.concatenate(
            [eu_ref[:, 16:18], jnp.zeros((_BE, 14), jnp.float32)], axis=1
        )


def _edge_mm(edge_attr, wT, b_row):
    return pl.pallas_call(
        _edge_mm_body,
        grid=(_NBE, 2),
        in_specs=[
            pl.BlockSpec((_BE, ED), lambda i, p: (i, 0)),
            pl.BlockSpec((ED, IN), lambda i, p: (0, 0)),
            pl.BlockSpec((1, IN), lambda i, p: (0, 0)),
        ],
        out_specs=pl.BlockSpec((_BE, 16), lambda i, p: (p * _NBE + i, 0)),
        out_shape=jax.ShapeDtypeStruct((2 * E, 16), jnp.float32),
        scratch_shapes=[pltpu.VMEM((_BE, IN), jnp.float32)],
    )(edge_attr, wT, b_row)


@functools.partial(
    pl.kernel,
    out_type=jax.ShapeDtypeStruct((2 * NP, 16), jnp.float32),
    mesh=_mesh,
    compiler_params=_sc_params,
    scratch_types=[
        pltpu.VMEM((K,), jnp.int32),
        pltpu.VMEM((K, 16), jnp.float32),
        pltpu.VMEM_SHARED((NP, 16), jnp.float32),
        pltpu.SemaphoreType.DMA,
        pltpu.SemaphoreType.DMA,
        pltpu.SemaphoreType.DMA,
    ],
)
def _sc_scatter_eu(eucat, src1, zz, pp, didx, rows, acc, sem_i, sem_a, sem_o):
    c = lax.axis_index("c")
    s = lax.axis_index("s")
    r0 = s * RPT
    pltpu.async_copy(zz, acc.at[pl.ds(r0, RPT), :], sem_o).wait()
    plsc.subcore_barrier()

    cpw = NCH // NTILE
    ch0 = s * cpw

    def body(i, carry):
        e0 = (ch0 + i) * K
        pltpu.async_copy(src1.at[pl.ds(e0, K)], didx, sem_i).wait()
        pltpu.async_copy(eucat.at[pl.ds(c * E + e0, K), :], rows, sem_i).wait()
        pltpu.async_copy(rows, acc.at[didx], sem_a, add=True).wait()
        return carry

    lax.fori_loop(0, cpw, body, 0)
    plsc.subcore_barrier()
    pltpu.async_copy(
        acc.at[pl.ds(r0, RPT), :], pp.at[pl.ds(c * NP + r0, RPT), :], sem_o
    ).wait()


_BC = 2000


def _combine_body(x_ref, plo_ref, phi_ref, lo_ref, hi_ref):
    lo_ref[...] = x_ref[:, 0:16] + plo_ref[...]
    hi_ref[...] = jnp.concatenate(
        [
            x_ref[:, 16:18] + phi_ref[:, 0:2],
            jnp.ones((_BC, 1), jnp.float32),
            jnp.zeros((_BC, 13), jnp.float32),
        ],
        axis=1,
    )


def _combine(x, plo, phi):
    return pl.pallas_call(
        _combine_body,
        grid=(N // _BC,),
        in_specs=[
            pl.BlockSpec((_BC, IN), lambda i: (i, 0)),
            pl.BlockSpec((_BC, 16), lambda i: (i, 0)),
            pl.BlockSpec((_BC, 16), lambda i: (i, 0)),
        ],
        out_specs=[
            pl.BlockSpec((_BC, 16), lambda i: (i, 0)),
            pl.BlockSpec((_BC, 16), lambda i: (i, 0)),
        ],
        out_shape=[
            jax.ShapeDtypeStruct((N, 16), jnp.float32),
            jax.ShapeDtypeStruct((N, 16), jnp.float32),
        ],
    )(x, plo, phi)


@functools.partial(
    pl.kernel,
    out_type=jax.ShapeDtypeStruct((2 * NP, 16), jnp.float32),
    mesh=_mesh,
    compiler_params=_sc_params,
    scratch_types=[
        pltpu.VMEM((K,), jnp.int32),
        pltpu.VMEM((K,), jnp.int32),
        pltpu.VMEM((K, 16), jnp.float32),
        pltpu.VMEM_SHARED((NP, 16), jnp.float32),
        pltpu.SemaphoreType.DMA,
        pltpu.SemaphoreType.DMA,
        pltpu.SemaphoreType.DMA,
        pltpu.SemaphoreType.DMA,
    ],
)
def _sc_agg1(x1cat, srccat, dst1, zz, aa,
             sidx, didx, rows, acc, sem_g, sem_i, sem_a, sem_o):
    c = lax.axis_index("c")
    s = lax.axis_index("s")
    r0 = s * RPT
    pltpu.async_copy(zz, acc.at[pl.ds(r0, RPT), :], sem_o).wait()
    plsc.subcore_barrier()

    cpw = NCH // NTILE
    ch0 = s * cpw

    def body(i, carry):
        e0 = (ch0 + i) * K
        pltpu.async_copy(srccat.at[pl.ds(c * E + e0, K)], sidx, sem_i).wait()
        pltpu.async_copy(dst1.at[pl.ds(e0, K)], didx, sem_i).wait()
        pltpu.async_copy(x1cat.at[sidx], rows, sem_g).wait()
        pltpu.async_copy(rows, acc.at[didx], sem_a, add=True).wait()
        return carry

    lax.fori_loop(0, cpw, body, 0)
    plsc.subcore_barrier()
    pltpu.async_copy(
        acc.at[pl.ds(r0, RPT), :], aa.at[pl.ds(c * NP + r0, RPT), :], sem_o
    ).wait()


def _conv1_body(alo_ref, ahi_ref, xlo_ref, xhi_ref, w1l_ref, w1r_ref, b1_ref,
                w2r_ref, h0_ref, h1_ref, h2_ref, h3_ref, hr_ref, deg_ref):
    deg = jnp.maximum(ahi_ref[:, 2:3], 1.0)
    mean = jnp.concatenate([alo_ref[...], ahi_ref[:, 0:2]], axis=1) / deg
    x1 = jnp.concatenate([xlo_ref[...], xhi_ref[:, 0:2]], axis=1)
    h = jnp.maximum(
        jnp.dot(mean, w1l_ref[...], preferred_element_type=jnp.float32)
        + jnp.dot(x1, w1r_ref[...], preferred_element_type=jnp.float32)
        + b1_ref[...],
        0.0,
    )
    h0_ref[...] = h[:, 0:16]
    h1_ref[...] = h[:, 16:32]
    h2_ref[...] = h[:, 32:48]
    h3_ref[...] = h[:, 48:64]
    hr_ref[...] = jnp.dot(h, w2r_ref[...], preferred_element_type=jnp.float32)
    deg_ref[...] = deg


def _conv1(alo, ahi, xlo, xhi, w1lT, w1rT, b1_row, w2rT):
    blk16 = pl.BlockSpec((_BC, 16), lambda i: (i, 0))
    return pl.pallas_call(
        _conv1_body,
        grid=(N // _BC,),
        in_specs=[
            blk16, blk16, blk16, blk16,
            pl.BlockSpec((IN, H), lambda i: (0, 0)),
            pl.BlockSpec((IN, H), lambda i: (0, 0)),
            pl.BlockSpec((1, H), lambda i: (0, 0)),
            pl.BlockSpec((H, H), lambda i: (0, 0)),
        ],
        out_specs=[
            blk16, blk16, blk16, blk16,
            pl.BlockSpec((_BC, H), lambda i: (i, 0)),
            pl.BlockSpec((_BC, 1), lambda i: (i, 0)),
        ],
        out_shape=[
            jax.ShapeDtypeStruct((N, 16), jnp.float32),
            jax.ShapeDtypeStruct((N, 16), jnp.float32),
            jax.ShapeDtypeStruct((N, 16), jnp.float32),
            jax.ShapeDtypeStruct((N, 16), jnp.float32),
            jax.ShapeDtypeStruct((N, H), jnp.float32),
            jax.ShapeDtypeStruct((N, 1), jnp.float32),
        ],
    )(alo, ahi, xlo, xhi, w1lT, w1rT, b1_row, w2rT)


@functools.partial(
    pl.kernel,
    out_type=jax.ShapeDtypeStruct((4 * NP, 16), jnp.float32),
    mesh=_mesh,
    compiler_params=_sc_params,
    scratch_types=[
        pltpu.VMEM((K,), jnp.int32),
        pltpu.VMEM((K,), jnp.int32),
        pltpu.VMEM((K, 16), jnp.float32),
        pltpu.VMEM_SHARED((NP, 16), jnp.float32),
        pltpu.SemaphoreType.DMA,
        pltpu.SemaphoreType.DMA,
        pltpu.SemaphoreType.DMA,
        pltpu.SemaphoreType.DMA,
    ],
)
def _sc_agg2(h1cat, srccat4, dst1, zz, oo,
             sidx, didx, rows, acc, sem_g, sem_i, sem_a, sem_o):
    c = lax.axis_index("c")
    s = lax.axis_index("s")
    r0 = s * RPT
    cpw = NCH // NTILE
    ch0 = s * cpw

    for jj in range(2):
        p = 2 * c + jj
        pltpu.async_copy(zz, acc.at[pl.ds(r0, RPT), :], sem_o).wait()
        plsc.subcore_barrier()

        def body(i, carry):
            e0 = (ch0 + i) * K
            pltpu.async_copy(
                srccat4.at[pl.ds(p * E + e0, K)], sidx, sem_i
            ).wait()
            pltpu.async_copy(dst1.at[pl.ds(e0, K)], didx, sem_i).wait()
            pltpu.async_copy(h1cat.at[sidx], rows, sem_g).wait()
            pltpu.async_copy(rows, acc.at[didx], sem_a, add=True).wait()
            return carry

        lax.fori_loop(0, cpw, body, 0)
        plsc.subcore_barrier()
        pltpu.async_copy(
            acc.at[pl.ds(r0, RPT), :], oo.at[pl.ds(p * NP + r0, RPT), :], sem_o
        ).wait()
        plsc.subcore_barrier()


def _pool_body(o0_ref, o1_ref, o2_ref, o3_ref, deg_ref, hr_ref, w2l_ref,
               b2_ref, batch_ref, pw_ref, pb_ref, out_ref, sums, cnt):
    i = pl.program_id(0)

    @pl.when(i == 0)
    def _():
        sums[...] = jnp.zeros_like(sums)
        cnt[...] = jnp.zeros_like(cnt)

    agg = jnp.concatenate(
        [o0_ref[...], o1_ref[...], o2_ref[...], o3_ref[...]], axis=1
    )
    mean = agg / deg_ref[...]
    h2 = jnp.maximum(
        jnp.dot(mean, w2l_ref[...], preferred_element_type=jnp.float32)
        + hr_ref[...]
        + b2_ref[...],
        0.0,
    )
    b = batch_ref[0, 0, :]
    onehot = (b[:, None] == lax.broadcasted_iota(jnp.int32, (_BC, G), 1)
              ).astype(jnp.float32)
    sums[...] += lax.dot_general(
        onehot, h2, (((0,), (0,)), ((), ())),
        preferred_element_type=jnp.float32,
    )
    cnt[...] += lax.dot_general(
        onehot, jnp.ones((_BC, 1), jnp.float32), (((0,), (0,)), ((), ())),
        preferred_element_type=jnp.float32,
    )

    @pl.when(i == (N // _BC) - 1)
    def _():
        pooled = sums[...] / jnp.maximum(cnt[...], 1.0)
        out_ref[...] = (
            jnp.dot(pooled, pw_ref[...], preferred_element_type=jnp.float32)
            + pb_ref[...]
        )


def _pool(o4, deg, hr1, w2lT, b2_row, batchr, projT, pb_row):
    blk16 = pl.BlockSpec((_BC, 16), lambda i: (i, 0))
    return pl.pallas_call(
        _pool_body,
        grid=(N // _BC,),
        in_specs=[
            blk16, blk16, blk16, blk16,
            pl.BlockSpec((_BC, 1), lambda i: (i, 0)),
            pl.BlockSpec((_BC, H), lambda i: (i, 0)),
            pl.BlockSpec((H, H), lambda i: (0, 0)),
            pl.BlockSpec((1, H), lambda i: (0, 0)),
            pl.BlockSpec((1, 1, _BC), lambda i: (i, 0, 0)),
            pl.BlockSpec((H, OUT), lambda i: (0, 0)),
            pl.BlockSpec((1, OUT), lambda i: (0, 0)),
        ],
        out_specs=pl.BlockSpec((G, OUT), lambda i: (0, 0)),
        out_shape=jax.ShapeDtypeStruct((G, OUT), jnp.float32),
        scratch_shapes=[
            pltpu.VMEM((G, H), jnp.float32),
            pltpu.VMEM((G, 1), jnp.float32),
        ],
    )(*o4, deg, hr1, w2lT, b2_row, batchr, projT, pb_row)


def kernel(x, edge_attr, edge_fc_w, edge_fc_b, w1l, w1r, b1, w2l, w2r, b2,
           proj_w, proj_b, edge_index, batch):
    src = edge_index[0]
    dst = edge_index[1]
    srccat = jnp.concatenate([src, src + N])
    srccat4 = jnp.concatenate([src, src + N, src + 2 * N, src + 3 * N])
    zz = jnp.zeros((RPT, 16), jnp.float32)

    eucat = _edge_mm(edge_attr, edge_fc_w.T, edge_fc_b.reshape(1, IN))
    pflat = _sc_scatter_eu(eucat, src, zz)
    x1lo, x1hi = _combine(x, pflat[:N], pflat[NP:NP + N])
    x1cat = jnp.concatenate([x1lo, x1hi], axis=0)
    aflat = _sc_agg1(x1cat, srccat, dst, zz)
    h1b0, h1b1, h1b2, h1b3, hr1, deg = _conv1(
        aflat[:N], aflat[NP:NP + N], x1lo, x1hi,
        w1l.T, w1r.T, b1.reshape(1, H), w2r.T,
    )
    h1cat = jnp.concatenate([h1b0, h1b1, h1b2, h1b3], axis=0)
    oflat = _sc_agg2(h1cat, srccat4, dst, zz)
    out = _pool(
        (oflat[:N], oflat[NP:NP + N], oflat[2 * NP:2 * NP + N],
         oflat[3 * NP:3 * NP + N]),
        deg, hr1, w2l.T, b2.reshape(1, H),
        batch.reshape(N // _BC, 1, _BC),
        proj_w.T, proj_b.reshape(1, OUT),
    )
    return out


# speedup vs baseline: 3.8660x; 2.1442x over previous
import functools

import jax
import jax.numpy as jnp
from jax import lax
from jax.experimental import pallas as pl
from jax.experimental.pallas import tpu as pltpu
from jax.experimental.pallas import tpu_sc as plsc

N = 100000
E = 1600000
IN = 18
ED = 22
H = 64
OUT = 128
G = 64

NTILE = 16
NP = 100096
RPT = NP // NTILE
K = 1000
NCH = E // K

_mesh = plsc.VectorSubcoreMesh(core_axis_name="c", subcore_axis_name="s")
_sc_params = pltpu.CompilerParams(use_tc_tiling_on_sc=False)


_BE = 8000
_NBE = E // _BE


def _edge_mm_body(attr_ref, w_ref, b_ref, out_ref, eu_ref):
    p = pl.program_id(1)

    @pl.when(p == 0)
    def _():
        eu_ref[...] = (
            jnp.dot(attr_ref[...], w_ref[...],
                    preferred_element_type=jnp.float32)
            + b_ref[...]
        )
        out_ref[...] = eu_ref[:, 0:16]

    @pl.when(p == 1)
    def _():
        out_ref[...] = jnp.concatenate(
            [eu_ref[:, 16:18], jnp.zeros((_BE, 14), jnp.float32)], axis=1
        )


def _edge_mm(edge_attr, wT, b_row):
    return pl.pallas_call(
        _edge_mm_body,
        grid=(_NBE, 2),
        in_specs=[
            pl.BlockSpec((_BE, ED), lambda i, p: (i, 0)),
            pl.BlockSpec((ED, IN), lambda i, p: (0, 0)),
            pl.BlockSpec((1, IN), lambda i, p: (0, 0)),
        ],
        out_specs=pl.BlockSpec((_BE, 16), lambda i, p: (p * _NBE + i, 0)),
        out_shape=jax.ShapeDtypeStruct((2 * E, 16), jnp.float32),
        scratch_shapes=[pltpu.VMEM((_BE, IN), jnp.float32)],
    )(edge_attr, wT, b_row)


@functools.partial(
    pl.kernel,
    out_type=jax.ShapeDtypeStruct((2 * NP, 16), jnp.float32),
    mesh=_mesh,
    compiler_params=_sc_params,
    scratch_types=[
        pltpu.VMEM((K,), jnp.int32),
        pltpu.VMEM((K, 16), jnp.float32),
        pltpu.VMEM_SHARED((NP, 16), jnp.float32),
        pltpu.SemaphoreType.DMA,
        pltpu.SemaphoreType.DMA,
        pltpu.SemaphoreType.DMA,
    ],
)
def _sc_scatter_eu(eucat, src1, zz, pp, didx, rows, acc, sem_i, sem_a, sem_o):
    c = lax.axis_index("c")
    s = lax.axis_index("s")
    r0 = s * RPT
    pltpu.async_copy(zz, acc.at[pl.ds(r0, RPT), :], sem_o).wait()
    plsc.subcore_barrier()

    cpw = NCH // NTILE
    ch0 = s * cpw

    def body(i, carry):
        e0 = (ch0 + i) * K
        pltpu.async_copy(src1.at[pl.ds(e0, K)], didx, sem_i).wait()
        pltpu.async_copy(eucat.at[pl.ds(c * E + e0, K), :], rows, sem_i).wait()
        pltpu.async_copy(rows, acc.at[didx], sem_a, add=True).wait()
        return carry

    lax.fori_loop(0, cpw, body, 0)
    plsc.subcore_barrier()
    pltpu.async_copy(
        acc.at[pl.ds(r0, RPT), :], pp.at[pl.ds(c * NP + r0, RPT), :], sem_o
    ).wait()


_BC = 2000


def _combine_body(x_ref, plo_ref, phi_ref, lo_ref, hi_ref):
    lo_ref[...] = x_ref[:, 0:16] + plo_ref[...]
    hi_ref[...] = jnp.concatenate(
        [
            x_ref[:, 16:18] + phi_ref[:, 0:2],
            jnp.ones((_BC, 1), jnp.float32),
            jnp.zeros((_BC, 13), jnp.float32),
        ],
        axis=1,
    )


def _combine(x, plo, phi):
    return pl.pallas_call(
        _combine_body,
        grid=(N // _BC,),
        in_specs=[
            pl.BlockSpec((_BC, IN), lambda i: (i, 0)),
            pl.BlockSpec((_BC, 16), lambda i: (i, 0)),
            pl.BlockSpec((_BC, 16), lambda i: (i, 0)),
        ],
        out_specs=[
            pl.BlockSpec((_BC, 16), lambda i: (i, 0)),
            pl.BlockSpec((_BC, 16), lambda i: (i, 0)),
        ],
        out_shape=[
            jax.ShapeDtypeStruct((N, 16), jnp.float32),
            jax.ShapeDtypeStruct((N, 16), jnp.float32),
        ],
    )(x, plo, phi)


@functools.partial(
    pl.kernel,
    out_type=jax.ShapeDtypeStruct((2 * NP, 16), jnp.float32),
    mesh=_mesh,
    compiler_params=_sc_params,
    scratch_types=[
        pltpu.VMEM((K,), jnp.int32),
        pltpu.VMEM((K,), jnp.int32),
        pltpu.VMEM((K, 16), jnp.float32),
        pltpu.VMEM_SHARED((NP, 16), jnp.float32),
        pltpu.SemaphoreType.DMA,
        pltpu.SemaphoreType.DMA,
        pltpu.SemaphoreType.DMA,
        pltpu.SemaphoreType.DMA,
    ],
)
def _sc_agg1(x1cat, srccat, dst1, zz, aa,
             sidx, didx, rows, acc, sem_g, sem_i, sem_a, sem_o):
    c = lax.axis_index("c")
    s = lax.axis_index("s")
    r0 = s * RPT
    pltpu.async_copy(zz, acc.at[pl.ds(r0, RPT), :], sem_o).wait()
    plsc.subcore_barrier()

    cpw = NCH // NTILE
    ch0 = s * cpw

    def body(i, carry):
        e0 = (ch0 + i) * K
        pltpu.async_copy(srccat.at[pl.ds(c * E + e0, K)], sidx, sem_i).wait()
        pltpu.async_copy(dst1.at[pl.ds(e0, K)], didx, sem_i).wait()
        pltpu.async_copy(x1cat.at[sidx], rows, sem_g).wait()
        pltpu.async_copy(rows, acc.at[didx], sem_a, add=True).wait()
        return carry

    lax.fori_loop(0, cpw, body, 0)
    plsc.subcore_barrier()
    pltpu.async_copy(
        acc.at[pl.ds(r0, RPT), :], aa.at[pl.ds(c * NP + r0, RPT), :], sem_o
    ).wait()


def _conv1_body(alo_ref, ahi_ref, xlo_ref, xhi_ref, w1l_ref, w1r_ref, b1_ref,
                w2r_ref, h0_ref, h1_ref, h2_ref, h3_ref, hr_ref, deg_ref):
    deg = jnp.maximum(ahi_ref[:, 2:3], 1.0)
    mean = jnp.concatenate([alo_ref[...], ahi_ref[:, 0:2]], axis=1) / deg
    x1 = jnp.concatenate([xlo_ref[...], xhi_ref[:, 0:2]], axis=1)
    h = jnp.maximum(
        jnp.dot(mean, w1l_ref[...], preferred_element_type=jnp.float32)
        + jnp.dot(x1, w1r_ref[...], preferred_element_type=jnp.float32)
        + b1_ref[...],
        0.0,
    )
    h0_ref[...] = h[:, 0:16]
    h1_ref[...] = h[:, 16:32]
    h2_ref[...] = h[:, 32:48]
    h3_ref[...] = h[:, 48:64]
    hr_ref[...] = jnp.dot(h, w2r_ref[...], preferred_element_type=jnp.float32)
    deg_ref[...] = deg


def _conv1(alo, ahi, xlo, xhi, w1lT, w1rT, b1_row, w2rT):
    blk16 = pl.BlockSpec((_BC, 16), lambda i: (i, 0))
    return pl.pallas_call(
        _conv1_body,
        grid=(N // _BC,),
        in_specs=[
            blk16, blk16, blk16, blk16,
            pl.BlockSpec((IN, H), lambda i: (0, 0)),
            pl.BlockSpec((IN, H), lambda i: (0, 0)),
            pl.BlockSpec((1, H), lambda i: (0, 0)),
            pl.BlockSpec((H, H), lambda i: (0, 0)),
        ],
        out_specs=[
            blk16, blk16, blk16, blk16,
            pl.BlockSpec((_BC, H), lambda i: (i, 0)),
            pl.BlockSpec((_BC, 1), lambda i: (i, 0)),
        ],
        out_shape=[
            jax.ShapeDtypeStruct((N, 16), jnp.float32),
            jax.ShapeDtypeStruct((N, 16), jnp.float32),
            jax.ShapeDtypeStruct((N, 16), jnp.float32),
            jax.ShapeDtypeStruct((N, 16), jnp.float32),
            jax.ShapeDtypeStruct((N, H), jnp.float32),
            jax.ShapeDtypeStruct((N, 1), jnp.float32),
        ],
    )(alo, ahi, xlo, xhi, w1lT, w1rT, b1_row, w2rT)


@functools.partial(
    pl.kernel,
    out_type=jax.ShapeDtypeStruct((4 * NP, 16), jnp.float32),
    mesh=_mesh,
    compiler_params=_sc_params,
    scratch_types=[
        pltpu.VMEM((K,), jnp.int32),
        pltpu.VMEM((K,), jnp.int32),
        pltpu.VMEM((K, 16), jnp.float32),
        pltpu.VMEM_SHARED((NP, 16), jnp.float32),
        pltpu.SemaphoreType.DMA,
        pltpu.SemaphoreType.DMA,
        pltpu.SemaphoreType.DMA,
        pltpu.SemaphoreType.DMA,
    ],
)
def _sc_agg2(h1cat, srccat4, dst1, zz, oo,
             sidx, didx, rows, acc, sem_g, sem_i, sem_a, sem_o):
    c = lax.axis_index("c")
    s = lax.axis_index("s")
    r0 = s * RPT
    cpw = NCH // NTILE
    ch0 = s * cpw

    for jj in range(2):
        p = 2 * c + jj
        pltpu.async_copy(zz, acc.at[pl.ds(r0, RPT), :], sem_o).wait()
        plsc.subcore_barrier()

        def body(i, carry):
            e0 = (ch0 + i) * K
            pltpu.async_copy(
                srccat4.at[pl.ds(p * E + e0, K)], sidx, sem_i
            ).wait()
            pltpu.async_copy(dst1.at[pl.ds(e0, K)], didx, sem_i).wait()
            pltpu.async_copy(h1cat.at[sidx], rows, sem_g).wait()
            pltpu.async_copy(rows, acc.at[didx], sem_a, add=True).wait()
            return carry

        lax.fori_loop(0, cpw, body, 0)
        plsc.subcore_barrier()
        pltpu.async_copy(
            acc.at[pl.ds(r0, RPT), :], oo.at[pl.ds(p * NP + r0, RPT), :], sem_o
        ).wait()
        plsc.subcore_barrier()


def _pool_body(o0_ref, o1_ref, o2_ref, o3_ref, deg_ref, hr_ref, w2l_ref,
               b2_ref, batch_ref, pw_ref, pb_ref, out_ref, sums, cnt):
    i = pl.program_id(0)

    @pl.when(i == 0)
    def _():
        sums[...] = jnp.zeros_like(sums)
        cnt[...] = jnp.zeros_like(cnt)

    agg = jnp.concatenate(
        [o0_ref[...], o1_ref[...], o2_ref[...], o3_ref[...]], axis=1
    )
    mean = agg / deg_ref[...]
    h2 = jnp.maximum(
        jnp.dot(mean, w2l_ref[...], preferred_element_type=jnp.float32)
        + hr_ref[...]
        + b2_ref[...],
        0.0,
    )
    b = batch_ref[0, 0, :]
    onehot = (b[:, None] == lax.broadcasted_iota(jnp.int32, (_BC, G), 1)
              ).astype(jnp.float32)
    sums[...] += lax.dot_general(
        onehot, h2, (((0,), (0,)), ((), ())),
        preferred_element_type=jnp.float32,
    )
    cnt[...] += lax.dot_general(
        onehot, jnp.ones((_BC, 1), jnp.float32), (((0,), (0,)), ((), ())),
        preferred_element_type=jnp.float32,
    )

    @pl.when(i == (N // _BC) - 1)
    def _():
        pooled = sums[...] / jnp.maximum(cnt[...], 1.0)
        out_ref[...] = (
            jnp.dot(pooled, pw_ref[...], preferred_element_type=jnp.float32)
            + pb_ref[...]
        )


def _pool(o4, deg, hr1, w2lT, b2_row, batchr, projT, pb_row):
    blk16 = pl.BlockSpec((_BC, 16), lambda i: (i, 0))
    return pl.pallas_call(
        _pool_body,
        grid=(N // _BC,),
        in_specs=[
            blk16, blk16, blk16, blk16,
            pl.BlockSpec((_BC, 1), lambda i: (i, 0)),
            pl.BlockSpec((_BC, H), lambda i: (i, 0)),
            pl.BlockSpec((H, H), lambda i: (0, 0)),
            pl.BlockSpec((1, H), lambda i: (0, 0)),
            pl.BlockSpec((1, 1, _BC), lambda i: (i, 0, 0)),
            pl.BlockSpec((H, OUT), lambda i: (0, 0)),
            pl.BlockSpec((1, OUT), lambda i: (0, 0)),
        ],
        out_specs=pl.BlockSpec((G, OUT), lambda i: (0, 0)),
        out_shape=jax.ShapeDtypeStruct((G, OUT), jnp.float32),
        scratch_shapes=[
            pltpu.VMEM((G, H), jnp.float32),
            pltpu.VMEM((G, 1), jnp.float32),
        ],
    )(*o4, deg, hr1, w2lT, b2_row, batchr, projT, pb_row)


def kernel(x, edge_attr, edge_fc_w, edge_fc_b, w1l, w1r, b1, w2l, w2r, b2,
           proj_w, proj_b, edge_index, batch):
    src = edge_index[0]
    dst = edge_index[1]
    srccat = jnp.concatenate([src, src + N])
    srccat4 = jnp.concatenate([src, src + N, src + 2 * N, src + 3 * N])
    zz = jnp.zeros((RPT, 16), jnp.float32)

    eucat = _edge_mm(edge_attr, edge_fc_w.T, edge_fc_b.reshape(1, IN))
    pflat = _sc_scatter_eu(eucat, src, zz)
    x1lo, x1hi = _combine(x, pflat[:N], pflat[NP:NP + N])
    x1cat = jnp.concatenate([x1lo, x1hi], axis=0)
    aflat = _sc_agg1(x1cat, srccat, dst, zz)
    h1b0, h1b1, h1b2, h1b3, hr1, deg = _conv1(
        aflat[:N], aflat[NP:NP + N], x1lo, x1hi,
        w1l.T, w1r.T, b1.reshape(1, H), w2r.T,
    )
    h1cat = jnp.concatenate([h1b0, h1b1, h1b2, h1b3], axis=0)
    oflat = _sc_agg2(h1cat, srccat4, dst, zz)
    out = _pool(
        (oflat[:N], oflat[NP:NP + N], oflat[2 * NP:2 * NP + N],
         oflat[3 * NP:3 * NP + N]),
        deg, hr1, w2l.T, b2.reshape(1, H),
        batch.reshape(N // _BC, 1, _BC),
        proj_w.T, proj_b.reshape(1, OUT),
    )
    return out


# speedup vs baseline: 6.5416x; 1.6921x over previous
import functools

import jax
import jax.numpy as jnp
from jax import lax
from jax.experimental import pallas as pl
from jax.experimental.pallas import tpu as pltpu
from jax.experimental.pallas import tpu_sc as plsc

N = 100000
E = 1600000
IN = 18
ED = 22
H = 64
OUT = 128
G = 64

NTILE = 16
NP = 100096
RPT = NP // NTILE
K = 800
NCH = E // K
CPW = NCH // NTILE

_mesh = plsc.VectorSubcoreMesh(core_axis_name="c", subcore_axis_name="s")
_sc_params = pltpu.CompilerParams(use_tc_tiling_on_sc=False)


def _pipeline(load_a, load_b, scat_a, scat_b):
    m = (CPW - 1) // 2 if CPW % 2 else CPW // 2 - 1

    load_a(0)

    def body(i, carry):
        j = 2 * i
        load_b(j + 1)
        scat_a(j)
        load_a(j + 2)
        scat_b(j + 1)
        return carry

    lax.fori_loop(0, m, body, 0)
    if CPW % 2:
        scat_a(2 * m)
    else:
        load_b(CPW - 1)
        scat_a(CPW - 2)
        scat_b(CPW - 1)


@functools.partial(
    pl.kernel,
    out_type=jax.ShapeDtypeStruct((2 * NP, 16), jnp.float32),
    mesh=_mesh,
    compiler_params=_sc_params,
    scratch_types=[
        pltpu.VMEM((K,), jnp.int32),
        pltpu.VMEM((K,), jnp.int32),
        pltpu.VMEM((K, 16), jnp.float32),
        pltpu.VMEM((K, 16), jnp.float32),
        pltpu.VMEM_SHARED((NP, 16), jnp.float32),
        pltpu.SemaphoreType.DMA,
        pltpu.SemaphoreType.DMA,
        pltpu.SemaphoreType.DMA,
        pltpu.SemaphoreType.DMA,
        pltpu.SemaphoreType.DMA,
    ],
)
def _sc_scatter_attr(attr24, src1, zz, pp, ia, ib, ra, rb, acc,
                     sem_ga, sem_gb, sem_i, sem_a, sem_o):
    c = lax.axis_index("c")
    s = lax.axis_index("s")
    r0 = s * RPT
    pltpu.async_copy(zz, acc.at[pl.ds(r0, RPT), :], sem_o).wait()
    plsc.subcore_barrier()
    ch0 = s * CPW
    col0 = c * 8

    def make_load(idx, rows, sem):
        def load(j):
            e0 = (ch0 + j) * K
            pltpu.async_copy(src1.at[pl.ds(e0, K)], idx, sem_i).wait()
            pltpu.async_copy(
                attr24.at[pl.ds(e0, K), pl.ds(col0, 16)], rows, sem
            )
        return load

    def make_scat(idx, rows, sem):
        def scat(j):
            e0 = (ch0 + j) * K
            pltpu.make_async_copy(
                attr24.at[pl.ds(e0, K), pl.ds(col0, 16)], rows, sem
            ).wait()
            pltpu.async_copy(rows, acc.at[idx], sem_a, add=True).wait()
        return scat

    _pipeline(make_load(ia, ra, sem_ga), make_load(ib, rb, sem_gb),
              make_scat(ia, ra, sem_ga), make_scat(ib, rb, sem_gb))

    plsc.subcore_barrier()
    pltpu.async_copy(
        acc.at[pl.ds(r0, RPT), :], pp.at[pl.ds(c * NP + r0, RPT), :], sem_o
    ).wait()


_BC = 2000


def _combine_body(x_ref, plo_ref, phi_ref, we_ref, be_ref, lo_ref, hi_ref):
    attr_agg = jnp.concatenate([plo_ref[...], phi_ref[:, 8:14]], axis=1)
    x1 = (
        x_ref[...]
        + jnp.dot(attr_agg, we_ref[...], preferred_element_type=jnp.float32)
        + phi_ref[:, 14:15] * be_ref[...]
    )
    lo_ref[...] = x1[:, 0:16]
    hi_ref[...] = jnp.concatenate(
        [
            x1[:, 16:18],
            jnp.ones((_BC, 1), jnp.float32),
            jnp.zeros((_BC, 13), jnp.float32),
        ],
        axis=1,
    )


def _combine(x, plo, phi, weT, be_row):
    return pl.pallas_call(
        _combine_body,
        grid=(N // _BC,),
        in_specs=[
            pl.BlockSpec((_BC, IN), lambda i: (i, 0)),
            pl.BlockSpec((_BC, 16), lambda i: (i, 0)),
            pl.BlockSpec((_BC, 16), lambda i: (i, 0)),
            pl.BlockSpec((ED, IN), lambda i: (0, 0)),
            pl.BlockSpec((1, IN), lambda i: (0, 0)),
        ],
        out_specs=[
            pl.BlockSpec((_BC, 16), lambda i: (i, 0)),
            pl.BlockSpec((_BC, 16), lambda i: (i, 0)),
        ],
        out_shape=[
            jax.ShapeDtypeStruct((N, 16), jnp.float32),
            jax.ShapeDtypeStruct((N, 16), jnp.float32),
        ],
    )(x, plo, phi, weT, be_row)


def _gs_pass(table, srcoff, dst1, acc, base, ch0,
             ia_s, ia_d, ib_s, ib_d, ra, rb, sem_ga, sem_gb, sem_i, sem_a):
    def make_load(si, di, rows, sem):
        def load(j):
            e0 = (ch0 + j) * K
            pltpu.async_copy(srcoff.at[pl.ds(base + e0, K)], si, sem_i).wait()
            pltpu.async_copy(dst1.at[pl.ds(e0, K)], di, sem_i).wait()
            pltpu.async_copy(table.at[si], rows, sem)
        return load

    def make_scat(si, di, rows, sem):
        def scat(j):
            pltpu.make_async_copy(table.at[si], rows, sem).wait()
            pltpu.async_copy(rows, acc.at[di], sem_a, add=True).wait()
        return scat

    _pipeline(make_load(ia_s, ia_d, ra, sem_ga),
              make_load(ib_s, ib_d, rb, sem_gb),
              make_scat(ia_s, ia_d, ra, sem_ga),
              make_scat(ib_s, ib_d, rb, sem_gb))


_GS_SCRATCH = [
    pltpu.VMEM((K,), jnp.int32),
    pltpu.VMEM((K,), jnp.int32),
    pltpu.VMEM((K,), jnp.int32),
    pltpu.VMEM((K,), jnp.int32),
    pltpu.VMEM((K, 16), jnp.float32),
    pltpu.VMEM((K, 16), jnp.float32),
    pltpu.VMEM_SHARED((NP, 16), jnp.float32),
    pltpu.SemaphoreType.DMA,
    pltpu.SemaphoreType.DMA,
    pltpu.SemaphoreType.DMA,
    pltpu.SemaphoreType.DMA,
    pltpu.SemaphoreType.DMA,
]


@functools.partial(
    pl.kernel,
    out_type=jax.ShapeDtypeStruct((2 * NP, 16), jnp.float32),
    mesh=_mesh,
    compiler_params=_sc_params,
    scratch_types=_GS_SCRATCH,
)
def _sc_agg1(x1cat, srccat, dst1, zz, aa,
             ia_s, ia_d, ib_s, ib_d, ra, rb, acc,
             sem_ga, sem_gb, sem_i, sem_a, sem_o):
    c = lax.axis_index("c")
    s = lax.axis_index("s")
    r0 = s * RPT
    pltpu.async_copy(zz, acc.at[pl.ds(r0, RPT), :], sem_o).wait()
    plsc.subcore_barrier()
    _gs_pass(x1cat, srccat, dst1, acc, c * E, s * CPW,
             ia_s, ia_d, ib_s, ib_d, ra, rb, sem_ga, sem_gb, sem_i, sem_a)
    plsc.subcore_barrier()
    pltpu.async_copy(
        acc.at[pl.ds(r0, RPT), :], aa.at[pl.ds(c * NP + r0, RPT), :], sem_o
    ).wait()


@functools.partial(
    pl.kernel,
    out_type=jax.ShapeDtypeStruct((4 * NP, 16), jnp.float32),
    mesh=_mesh,
    compiler_params=_sc_params,
    scratch_types=_GS_SCRATCH,
)
def _sc_agg2(h1cat, srccat4, dst1, zz, oo,
             ia_s, ia_d, ib_s, ib_d, ra, rb, acc,
             sem_ga, sem_gb, sem_i, sem_a, sem_o):
    c = lax.axis_index("c")
    s = lax.axis_index("s")
    r0 = s * RPT

    for jj in range(2):
        p = 2 * c + jj
        pltpu.async_copy(zz, acc.at[pl.ds(r0, RPT), :], sem_o).wait()
        plsc.subcore_barrier()
        _gs_pass(h1cat, srccat4, dst1, acc, p * E, s * CPW,
                 ia_s, ia_d, ib_s, ib_d, ra, rb, sem_ga, sem_gb, sem_i, sem_a)
        plsc.subcore_barrier()
        pltpu.async_copy(
            acc.at[pl.ds(r0, RPT), :], oo.at[pl.ds(p * NP + r0, RPT), :],
            sem_o,
        ).wait()
        plsc.subcore_barrier()


def _conv1_body(alo_ref, ahi_ref, xlo_ref, xhi_ref, w1l_ref, w1r_ref, b1_ref,
                w2r_ref, h0_ref, h1_ref, h2_ref, h3_ref, hr_ref, deg_ref):
    deg = jnp.maximum(ahi_ref[:, 2:3], 1.0)
    mean = jnp.concatenate([alo_ref[...], ahi_ref[:, 0:2]], axis=1) / deg
    x1 = jnp.concatenate([xlo_ref[...], xhi_ref[:, 0:2]], axis=1)
    h = jnp.maximum(
        jnp.dot(mean, w1l_ref[...], preferred_element_type=jnp.float32)
        + jnp.dot(x1, w1r_ref[...], preferred_element_type=jnp.float32)
        + b1_ref[...],
        0.0,
    )
    h0_ref[...] = h[:, 0:16]
    h1_ref[...] = h[:, 16:32]
    h2_ref[...] = h[:, 32:48]
    h3_ref[...] = h[:, 48:64]
    hr_ref[...] = jnp.dot(h, w2r_ref[...], preferred_element_type=jnp.float32)
    deg_ref[...] = deg


def _conv1(alo, ahi, xlo, xhi, w1lT, w1rT, b1_row, w2rT):
    blk16 = pl.BlockSpec((_BC, 16), lambda i: (i, 0))
    return pl.pallas_call(
        _conv1_body,
        grid=(N // _BC,),
        in_specs=[
            blk16, blk16, blk16, blk16,
            pl.BlockSpec((IN, H), lambda i: (0, 0)),
            pl.BlockSpec((IN, H), lambda i: (0, 0)),
            pl.BlockSpec((1, H), lambda i: (0, 0)),
            pl.BlockSpec((H, H), lambda i: (0, 0)),
        ],
        out_specs=[
            blk16, blk16, blk16, blk16,
            pl.BlockSpec((_BC, H), lambda i: (i, 0)),
            pl.BlockSpec((_BC, 1), lambda i: (i, 0)),
        ],
        out_shape=[
            jax.ShapeDtypeStruct((N, 16), jnp.float32),
            jax.ShapeDtypeStruct((N, 16), jnp.float32),
            jax.ShapeDtypeStruct((N, 16), jnp.float32),
            jax.ShapeDtypeStruct((N, 16), jnp.float32),
            jax.ShapeDtypeStruct((N, H), jnp.float32),
            jax.ShapeDtypeStruct((N, 1), jnp.float32),
        ],
    )(alo, ahi, xlo, xhi, w1lT, w1rT, b1_row, w2rT)


def _pool_body(o0_ref, o1_ref, o2_ref, o3_ref, deg_ref, hr_ref, w2l_ref,
               b2_ref, batch_ref, pw_ref, pb_ref, out_ref, sums, cnt):
    i = pl.program_id(0)

    @pl.when(i == 0)
    def _():
        sums[...] = jnp.zeros_like(sums)
        cnt[...] = jnp.zeros_like(cnt)

    agg = jnp.concatenate(
        [o0_ref[...], o1_ref[...], o2_ref[...], o3_ref[...]], axis=1
    )
    mean = agg / deg_ref[...]
    h2 = jnp.maximum(
        jnp.dot(mean, w2l_ref[...], preferred_element_type=jnp.float32)
        + hr_ref[...]
        + b2_ref[...],
        0.0,
    )
    b = batch_ref[0, 0, :]
    onehot = (b[:, None] == lax.broadcasted_iota(jnp.int32, (_BC, G), 1)
              ).astype(jnp.float32)
    sums[...] += lax.dot_general(
        onehot, h2, (((0,), (0,)), ((), ())),
        preferred_element_type=jnp.float32,
    )
    cnt[...] += lax.dot_general(
        onehot, jnp.ones((_BC, 1), jnp.float32), (((0,), (0,)), ((), ())),
        preferred_element_type=jnp.float32,
    )

    @pl.when(i == (N // _BC) - 1)
    def _():
        pooled = sums[...] / jnp.maximum(cnt[...], 1.0)
        out_ref[...] = (
            jnp.dot(pooled, pw_ref[...], preferred_element_type=jnp.float32)
            + pb_ref[...]
        )


def _pool(o4, deg, hr1, w2lT, b2_row, batchr, projT, pb_row):
    blk16 = pl.BlockSpec((_BC, 16), lambda i: (i, 0))
    return pl.pallas_call(
        _pool_body,
        grid=(N // _BC,),
        in_specs=[
            blk16, blk16, blk16, blk16,
            pl.BlockSpec((_BC, 1), lambda i: (i, 0)),
            pl.BlockSpec((_BC, H), lambda i: (i, 0)),
            pl.BlockSpec((H, H), lambda i: (0, 0)),
            pl.BlockSpec((1, H), lambda i: (0, 0)),
            pl.BlockSpec((1, 1, _BC), lambda i: (i, 0, 0)),
            pl.BlockSpec((H, OUT), lambda i: (0, 0)),
            pl.BlockSpec((1, OUT), lambda i: (0, 0)),
        ],
        out_specs=pl.BlockSpec((G, OUT), lambda i: (0, 0)),
        out_shape=jax.ShapeDtypeStruct((G, OUT), jnp.float32),
        scratch_shapes=[
            pltpu.VMEM((G, H), jnp.float32),
            pltpu.VMEM((G, 1), jnp.float32),
        ],
    )(*o4, deg, hr1, w2lT, b2_row, batchr, projT, pb_row)


def kernel(x, edge_attr, edge_fc_w, edge_fc_b, w1l, w1r, b1, w2l, w2r, b2,
           proj_w, proj_b, edge_index, batch):
    src = edge_index[0]
    dst = edge_index[1]
    srccat = jnp.concatenate([src, src + N])
    srccat4 = jnp.concatenate([src, src + N, src + 2 * N, src + 3 * N])
    zz = jnp.zeros((RPT, 16), jnp.float32)
    attr24 = jnp.concatenate(
        [edge_attr, jnp.ones((E, 1), jnp.float32),
         jnp.zeros((E, 1), jnp.float32)], axis=1)

    pflat = _sc_scatter_attr(attr24, src, zz)
    x1lo, x1hi = _combine(x, pflat[:N], pflat[NP:NP + N],
                          edge_fc_w.T, edge_fc_b.reshape(1, IN))
    x1cat = jnp.concatenate([x1lo, x1hi], axis=0)
    aflat = _sc_agg1(x1cat, srccat, dst, zz)
    h1b0, h1b1, h1b2, h1b3, hr1, deg = _conv1(
        aflat[:N], aflat[NP:NP + N], x1lo, x1hi,
        w1l.T, w1r.T, b1.reshape(1, H), w2r.T,
    )
    h1cat = jnp.concatenate([h1b0, h1b1, h1b2, h1b3], axis=0)
    oflat = _sc_agg2(h1cat, srccat4, dst, zz)
    out = _pool(
        (oflat[:N], oflat[NP:NP + N], oflat[2 * NP:2 * NP + N],
         oflat[3 * NP:3 * NP + N]),
        deg, hr1, w2l.T, b2.reshape(1, H),
        batch.reshape(N // _BC, 1, _BC),
        proj_w.T, proj_b.reshape(1, OUT),
    )
    return out
